# Initial kernel scaffold; baseline (speedup 1.0000x reference)
import jax
import jax.numpy as jnp
from jax.experimental import pallas as pl


def kernel(x, edge_index, edge_weight, W1, b1, W2, b2, w_p0, b_p0, w_p1, b_p1, w_v, b_v):
    raise NotImplementedError("write your pallas kernel here")


# speedup vs baseline: 2.9288x; 2.9288x over previous
import functools

import jax
import jax.numpy as jnp
from jax import lax
from jax.experimental import pallas as pl
from jax.experimental.pallas import tpu as pltpu
from jax.experimental.pallas import tpu_sc as plsc

L = 16
NC = 2
NS = 16
NW = NC * NS
CH = 128
DB = 128


def _sc_edge_aggregate(n_nodes, n_chunks_per_worker):
    N = n_nodes
    CPW = n_chunks_per_worker
    ZR = 80
    NBLK = N // ZR

    mesh = plsc.VectorSubcoreMesh(core_axis_name="c", subcore_axis_name="s",
                                  num_cores=NC, num_subcores=NS)

    @functools.partial(
        pl.kernel,
        out_type=jax.ShapeDtypeStruct((NC, N, DB), jnp.float32),
        mesh=mesh,
        scratch_types=[
            pltpu.VMEM((CH,), jnp.int32),
            pltpu.VMEM((CH,), jnp.int32),
            pltpu.VMEM((CH,), jnp.float32),
            pltpu.VMEM((CH, DB), jnp.float32),
            pltpu.VMEM_SHARED((N, DB), jnp.float32),
            pltpu.SemaphoreType.DMA,
        ],
    )
    def k(table_h, src_h, dst_h, ew_h, out_h,
          srcv, dstv, ewv, rows, acc, sem):
        c = lax.axis_index("c")
        s = lax.axis_index("s")
        wid = s * NC + c

        def zrow(i, carry):
            for kk in range(DB // L):
                rows[i, pl.ds(kk * L, L)] = jnp.zeros((L,), jnp.float32)
            return carry
        lax.fori_loop(0, ZR, zrow, 0)

        def zblk(j, carry):
            blk = s + NS * j

            @pl.when(blk < NBLK)
            def _():
                off = pl.multiple_of(blk * ZR, 8)
                pltpu.sync_copy(rows.at[pl.ds(0, ZR)], acc.at[pl.ds(off, ZR)])
            return carry
        lax.fori_loop(0, -(-NBLK // NS), zblk, 0)

        plsc.subcore_barrier()

        def chunk(i, carry):
            base = pl.multiple_of((wid * CPW + i) * CH, 8)
            pltpu.sync_copy(src_h.at[pl.ds(base, CH)], srcv)
            pltpu.sync_copy(dst_h.at[pl.ds(base, CH)], dstv)
            pltpu.sync_copy(ew_h.at[pl.ds(base, CH)], ewv)

            pltpu.async_copy(table_h.at[srcv], rows, sem).wait()

            def grp(g, gcarry):
                w16 = ewv[pl.ds(g * L, L)]
                for l in range(L):
                    e = g * L + l
                    w = jnp.full((L,), w16[l])
                    for kk in range(DB // L):
                        rows[e, pl.ds(kk * L, L)] = (
                            rows[e, pl.ds(kk * L, L)] * w)
                return gcarry
            lax.fori_loop(0, CH // L, grp, 0)

            pltpu.sync_copy(rows, acc.at[dstv], add=True)
            return carry
        lax.fori_loop(0, CPW, chunk, 0)

        plsc.subcore_barrier()

        def oblk(j, carry):
            blk = s + NS * j

            @pl.when(blk < NBLK)
            def _():
                off = pl.multiple_of(blk * ZR, 8)
                pltpu.sync_copy(acc.at[pl.ds(off, ZR)],
                                out_h.at[c, pl.ds(off, ZR)])
            return carry
        lax.fori_loop(0, -(-NBLK // NS), oblk, 0)

    return k


def _layer1_dense(agg, W, b):
    N = agg.shape[1]
    K = agg.shape[2]
    H = W.shape[1]
    BN = 1000

    def body(a_ref, w_ref, b_ref, o0_ref, o1_ref):
        a = a_ref[0] + a_ref[1]
        h = jnp.dot(a, w_ref[...], preferred_element_type=jnp.float32)
        h = jnp.maximum(h + b_ref[...], 0.0)
        o0_ref[...] = h[:, :H // 2]
        o1_ref[...] = h[:, H // 2:]

    return pl.pallas_call(
        body,
        grid=(N // BN,),
        in_specs=[
            pl.BlockSpec((NC, BN, K), lambda i: (0, i, 0)),
            pl.BlockSpec((K, H), lambda i: (0, 0)),
            pl.BlockSpec((1, H), lambda i: (0, 0)),
        ],
        out_specs=[
            pl.BlockSpec((BN, H // 2), lambda i: (i, 0)),
            pl.BlockSpec((BN, H // 2), lambda i: (i, 0)),
        ],
        out_shape=[
            jax.ShapeDtypeStruct((N, H // 2), jnp.float32),
            jax.ShapeDtypeStruct((N, H // 2), jnp.float32),
        ],
    )(agg, W, b.reshape(1, H))


def _layer2_dense_heads(agg, W, b, wp, bp):
    N = agg.shape[1]
    K = agg.shape[2]
    H = W.shape[1]
    BN = 1000

    def body(a_ref, w_ref, b_ref, wp_ref, bp_ref,
             pi0_ref, pi1_ref, val_ref, accs_ref):
        i = pl.program_id(0)
        a = a_ref[0] + a_ref[1]
        h = jnp.dot(a, w_ref[...], preferred_element_type=jnp.float32)
        h = jnp.maximum(h + b_ref[...], 0.0)
        wpv = wp_ref[...]
        norm = jnp.sqrt(jnp.sum(wpv * wpv, axis=0, keepdims=True)) + 1e-12
        wpn = wpv / norm
        P = jnp.dot(h, wpn, preferred_element_type=jnp.float32)
        pi0_ref[...] = P[:, 0:1] + bp_ref[0, 0]
        pi1_ref[...] = P[:, 1:2] + bp_ref[0, 1]

        @pl.when(i == 0)
        def _():
            accs_ref[0, 0] = 0.0
        tot = accs_ref[0, 0] + jnp.sum(P[:, 2])
        accs_ref[0, 0] = tot
        val_ref[...] = jnp.reshape(tot / N + bp_ref[0, 2], (1, 1))

    return pl.pallas_call(
        body,
        grid=(N // BN,),
        in_specs=[
            pl.BlockSpec((NC, BN, K), lambda i: (0, i, 0)),
            pl.BlockSpec((K, H), lambda i: (0, 0)),
            pl.BlockSpec((1, H), lambda i: (0, 0)),
            pl.BlockSpec((H, 3), lambda i: (0, 0)),
            pl.BlockSpec((1, 3), lambda i: (0, 0)),
        ],
        out_specs=[
            pl.BlockSpec((BN, 1), lambda i: (i, 0)),
            pl.BlockSpec((BN, 1), lambda i: (i, 0)),
            pl.BlockSpec((1, 1), lambda i: (0, 0)),
        ],
        out_shape=[
            jax.ShapeDtypeStruct((N, 1), jnp.float32),
            jax.ShapeDtypeStruct((N, 1), jnp.float32),
            jax.ShapeDtypeStruct((1, 1), jnp.float32),
        ],
        scratch_shapes=[pltpu.SMEM((1, 1), jnp.float32)],
    )(agg, W, b.reshape(1, H), wp, bp)


def kernel(x, edge_index, edge_weight, W1, b1, W2, b2,
           w_p0, b_p0, w_p1, b_p1, w_v, b_v):
    N, D = x.shape
    E = edge_index.shape[1]
    H = W1.shape[1]

    CPW = -(-E // (NW * CH))
    EP = NW * CPW * CH
    pad = EP - E
    src = jnp.concatenate([edge_index[0], jnp.zeros((pad,), jnp.int32)])
    dst = jnp.concatenate([edge_index[1], jnp.zeros((pad,), jnp.int32)])
    ew = jnp.concatenate([edge_weight, jnp.zeros((pad,), jnp.float32)])

    sc_agg = _sc_edge_aggregate(N, CPW)

    agg1 = sc_agg(x, src, dst, ew)
    h1a, h1b = _layer1_dense(agg1, W1, b1)

    agg2a = sc_agg(h1a, src, dst, ew)
    agg2b = sc_agg(h1b, src, dst, ew)
    agg2 = jnp.concatenate([agg2a, agg2b], axis=2)

    wp = jnp.concatenate([w_p0, w_p1, w_v], axis=1)
    bp = jnp.stack([b_p0, b_p1, b_v], axis=1)
    pi0, pi1, value = _layer2_dense_heads(agg2, W2, b2, wp, bp)

    pi = jnp.concatenate([pi0, pi1], axis=0)
    return (pi, value)
